# baseline (speedup 1.0000x reference)
import jax
import jax.numpy as jnp
from jax import lax
from jax.experimental import pallas as pl
from jax.experimental.pallas import tpu as pltpu
from jax.experimental.pallas import tpu_sc as plsc

N = 10000
E = 160000
D_IN = 256
D_H = 512
D_ACT = 4

NP = 10240
NC, NS, L = 2, 16, 16
NW = NC * NS
KB = 128
NB_E = 40
EW = NB_E * KB
EP = EW * NW
NPW = NP // NS
CHUNKS = D_H // 128
MESH = plsc.VectorSubcoreMesh(core_axis_name="c", subcore_axis_name="s")


def _wid():
    return lax.axis_index("s") * NC + lax.axis_index("c")


def _hist_body(dst_hbm, out_hbm, dstv, histv, resv, tmpv, acc_sh):
    core = lax.axis_index("c")
    sub = lax.axis_index("s")
    wid = _wid()
    pltpu.sync_copy(dst_hbm.at[wid], dstv)

    @pl.loop(0, NP // L)
    def _zero(j):
        histv[pl.ds(j * L, L)] = jnp.zeros((L,), jnp.float32)

    ones = jnp.ones((L,), jnp.float32)
    lane = lax.iota(jnp.int32, L)

    @pl.loop(0, NB_E)
    def _batch(i):
        base = wid * EW + i * KB
        for j in range(KB // L):
            idx = dstv[i, pl.ds(j * L, L)]
            valid = (base + j * L + lane) < E
            plsc.addupdate_scatter(histv, [idx], ones, mask=valid)

    pltpu.sync_copy(histv, acc_sh.at[sub])
    plsc.subcore_barrier()

    pltpu.sync_copy(acc_sh.at[:, pl.ds(sub * NPW, NPW)], tmpv)

    @pl.loop(0, NPW // L)
    def _comb(j):
        s = tmpv[0, pl.ds(j * L, L)]
        for t in range(1, NS):
            s = s + tmpv[t, pl.ds(j * L, L)]
        resv[pl.ds(j * L, L)] = s

    pltpu.sync_copy(resv, out_hbm.at[core, pl.ds(sub * NPW, NPW)])


_hist = pl.kernel(
    _hist_body,
    out_type=jax.ShapeDtypeStruct((NC, NP), jnp.float32),
    mesh=MESH,
    compiler_params=pltpu.CompilerParams(needs_layout_passes=False),
    scratch_types=[
        pltpu.VMEM((NB_E, KB), jnp.int32),
        pltpu.VMEM((NP,), jnp.float32),
        pltpu.VMEM((NPW,), jnp.float32),
        pltpu.VMEM((NS, NPW), jnp.float32),
        pltpu.VMEM_SHARED((NS, NP), jnp.float32),
    ],
)


def _spmm_body(h_hbm, src_hbm, dst_hbm, out_hbm,
               srcv, dstv, gidx0, gidx1, rows0, rows1, zbuf, acc_sh,
               sem0, sem1):
    core = lax.axis_index("c")
    sub = lax.axis_index("s")
    gidx = [gidx0, gidx1]
    rows = [rows0, rows1]
    sems = [sem0, sem1]

    @pl.loop(0, L)
    def _z(i):
        for g in range(KB // L):
            zbuf[i, pl.ds(g * L, L)] = jnp.zeros((L,), jnp.float32)

    def _start_gather(b, i, off):
        for j in range(KB // L):
            gidx[b][pl.ds(j * L, L)] = srcv[i, pl.ds(j * L, L)] + off
        pltpu.async_copy(h_hbm.at[gidx[b]], rows[b], sems[b])

    for cc_local in range(2):
        cc = core * 2 + cc_local

        @pl.loop(0, NPW // L)
        def _zacc(j):
            pltpu.sync_copy(zbuf, acc_sh.at[pl.ds(sub * NPW + j * L, L)])

        plsc.subcore_barrier()

        for sl in range(2):
            pltpu.sync_copy(src_hbm.at[sub * 2 + sl], srcv)
            pltpu.sync_copy(dst_hbm.at[sub * 2 + sl], dstv)
            off = cc * NP
            _start_gather(0, 0, off)
            _start_gather(1, 1, off)

            @pl.loop(0, NB_E, step=2)
            def _batch(i):
                for b in range(2):
                    pltpu.make_async_copy(h_hbm.at[gidx[b]], rows[b],
                                          sems[b]).wait()
                    pltpu.sync_copy(rows[b], acc_sh.at[dstv.at[i + b]],
                                    add=True)

                    @pl.when(i + b + 2 < NB_E)
                    def _():
                        _start_gather(b, i + b + 2, off)

        plsc.subcore_barrier()
        pltpu.sync_copy(acc_sh.at[pl.ds(sub * NPW, NPW)],
                        out_hbm.at[cc, pl.ds(sub * NPW, NPW)])


_spmm = pl.kernel(
    _spmm_body,
    out_type=jax.ShapeDtypeStruct((CHUNKS, NP, 128), jnp.float32),
    mesh=MESH,
    compiler_params=pltpu.CompilerParams(needs_layout_passes=False),
    scratch_types=[
        pltpu.VMEM((NB_E, KB), jnp.int32),
        pltpu.VMEM((NB_E, KB), jnp.int32),
        pltpu.VMEM((KB,), jnp.int32),
        pltpu.VMEM((KB,), jnp.int32),
        pltpu.VMEM((KB, 128), jnp.float32),
        pltpu.VMEM((KB, 128), jnp.float32),
        pltpu.VMEM((L, 128), jnp.float32),
        pltpu.VMEM_SHARED((NP, 128), jnp.float32),
        pltpu.SemaphoreType.DMA,
        pltpu.SemaphoreType.DMA,
    ],
)


BN = 512


def _gemm1_body(x_ref, w_ref, dv_ref, o_ref):
    o_ref[0] = jnp.dot(x_ref[...], w_ref[...],
                       preferred_element_type=jnp.float32) * dv_ref[...]


def _gemm1(x, w, dinv_r):
    return pl.pallas_call(
        _gemm1_body,
        grid=(CHUNKS, NP // BN),
        in_specs=[
            pl.BlockSpec((BN, D_IN), lambda i, j: (j, 0)),
            pl.BlockSpec((D_IN, 128), lambda i, j: (0, i)),
            pl.BlockSpec((BN, 1), lambda i, j: (j, 0)),
        ],
        out_specs=pl.BlockSpec((1, BN, 128), lambda i, j: (i, j, 0)),
        out_shape=jax.ShapeDtypeStruct((CHUNKS, NP, 128), jnp.float32),
    )(x, w, dinv_r)


def _gemm2_body(a_ref, h_ref, b_ref, w_ref, dv_ref, o_ref):
    @pl.when(pl.program_id(2) == 0)
    def _():
        o_ref[0] = jnp.zeros_like(o_ref[0])

    act = jnp.maximum((a_ref[0] + h_ref[0]) * dv_ref[...] + b_ref[0], 0.0)
    o_ref[0] += jnp.dot(act, w_ref[...], preferred_element_type=jnp.float32)

    @pl.when(pl.program_id(2) == CHUNKS - 1)
    def _():
        o_ref[0] = o_ref[0] * dv_ref[...]


def _gemm2(agg, h, brow, w, dinv_r):
    return pl.pallas_call(
        _gemm2_body,
        grid=(CHUNKS, NP // BN, CHUNKS),
        in_specs=[
            pl.BlockSpec((1, BN, 128), lambda i, j, k: (k, j, 0)),
            pl.BlockSpec((1, BN, 128), lambda i, j, k: (k, j, 0)),
            pl.BlockSpec((1, 1, 128), lambda i, j, k: (k, 0, 0)),
            pl.BlockSpec((128, 128), lambda i, j, k: (k, i)),
            pl.BlockSpec((BN, 1), lambda i, j, k: (j, 0)),
        ],
        out_specs=pl.BlockSpec((1, BN, 128), lambda i, j, k: (i, j, 0)),
        out_shape=jax.ShapeDtypeStruct((CHUNKS, NP, 128), jnp.float32),
    )(agg, h, brow, w, dinv_r)


def _head_body(agg_ref, h_ref, dv_ref, b2_ref, wp1_ref, bp1_ref, wp2_ref,
               bp2_ref, o_ref):
    a = jnp.concatenate([agg_ref[c] + h_ref[c] for c in range(CHUNKS)],
                        axis=-1)
    a = jnp.maximum(a * dv_ref[...] + b2_ref[...], 0.0)
    p = jnp.maximum(jnp.dot(a, wp1_ref[...],
                            preferred_element_type=jnp.float32) + bp1_ref[...],
                    0.0)
    lg = jnp.dot(p, wp2_ref[...],
                 preferred_element_type=jnp.float32) + bp2_ref[...]
    m = jnp.max(lg, axis=-1, keepdims=True)
    ex = jnp.exp(lg - m)
    o_ref[...] = ex / jnp.sum(ex, axis=-1, keepdims=True)


def _head(agg, h, dinv_r, b2row, wp1, bp1row, wp2p, bp2p):
    return pl.pallas_call(
        _head_body,
        grid=(NP // BN,),
        in_specs=[
            pl.BlockSpec((CHUNKS, BN, 128), lambda j: (0, j, 0)),
            pl.BlockSpec((CHUNKS, BN, 128), lambda j: (0, j, 0)),
            pl.BlockSpec((BN, 1), lambda j: (j, 0)),
            pl.BlockSpec((1, D_H), lambda j: (0, 0)),
            pl.BlockSpec((D_H, D_H), lambda j: (0, 0)),
            pl.BlockSpec((1, D_H), lambda j: (0, 0)),
            pl.BlockSpec((D_H, 128), lambda j: (0, 0)),
            pl.BlockSpec((1, 128), lambda j: (0, 0)),
        ],
        out_specs=pl.BlockSpec((BN, 128), lambda j: (j, 0)),
        out_shape=jax.ShapeDtypeStruct((NP, 128), jnp.float32),
    )(agg, h, dinv_r, b2row, wp1, bp1row, wp2p, bp2p)


@jax.jit
def _run(x, edge_index, W1, b1, W2, b2, Wp1, bp1, Wp2, bp2):
    src = edge_index[0].astype(jnp.int32)
    dst = edge_index[1].astype(jnp.int32)
    srcf = jnp.pad(src, (0, EP - E), constant_values=NP - 1)
    dstf = jnp.pad(dst, (0, EP - E), constant_values=NP - 1)
    src3 = srcf.reshape(NW, NB_E, KB)
    dst3 = dstf.reshape(NW, NB_E, KB)

    degp = _hist(dst3)
    deg = degp[0] + degp[1] + 1.0
    dinv_r = lax.rsqrt(deg).reshape(NP, 1)

    x_pad = jnp.pad(x, ((0, NP - N), (0, 0)))
    h1 = _gemm1(x_pad, W1, dinv_r)
    agg1 = _spmm(h1.reshape(CHUNKS * NP, 128), src3, dst3)
    h2 = _gemm2(agg1, h1, b1.reshape(CHUNKS, 1, 128), W2, dinv_r)
    agg2 = _spmm(h2.reshape(CHUNKS * NP, 128), src3, dst3)

    wp2p = jnp.pad(Wp2, ((0, 0), (0, 128 - D_ACT)))
    bp2p = jnp.pad(bp2, (0, 128 - D_ACT), constant_values=-1e30)
    acts = _head(agg2, h2, dinv_r, b2.reshape(1, D_H), Wp1,
                 bp1.reshape(1, D_H),
                 wp2p, bp2p.reshape(1, 128))
    return acts[:N, :D_ACT]


def kernel(x, edge_index, W1, b1, W2, b2, Wp1, bp1, Wp2, bp2):
    return _run(x, edge_index, W1, b1, W2, b2, Wp1, bp1, Wp2, bp2)


# speedup vs baseline: 1.9092x; 1.9092x over previous
import jax
import jax.numpy as jnp
from jax import lax
from jax.experimental import pallas as pl
from jax.experimental.pallas import tpu as pltpu
from jax.experimental.pallas import tpu_sc as plsc

N = 10000
E = 160000
D_IN = 256
D_H = 512
D_ACT = 4

NP = 10240
NC, NS, L = 2, 16, 16
NW = NC * NS
KB = 128
NB_E = 40
EW = NB_E * KB
EP = EW * NW
ES = E // NW
NPW = NP // NS
CHUNKS = D_H // 128
MESH = plsc.VectorSubcoreMesh(core_axis_name="c", subcore_axis_name="s")


def _wid():
    return lax.axis_index("s") * NC + lax.axis_index("c")


def _hist_body(dst_hbm, out_hbm, dstv, histv, resv, tmpv, acc_sh):
    core = lax.axis_index("c")
    sub = lax.axis_index("s")
    wid = _wid()
    pltpu.sync_copy(dst_hbm.at[wid], dstv)

    @pl.loop(0, NP // L)
    def _zero(j):
        histv[pl.ds(j * L, L)] = jnp.zeros((L,), jnp.float32)

    ones = jnp.ones((L,), jnp.float32)
    lane = lax.iota(jnp.int32, L)

    @pl.loop(0, NB_E)
    def _batch(i):
        base = i * KB
        for j in range(KB // L):
            idx = dstv[i, pl.ds(j * L, L)]
            valid = (base + j * L + lane) < ES
            plsc.addupdate_scatter(histv, [idx], ones, mask=valid)

    pltpu.sync_copy(histv, acc_sh.at[sub])
    plsc.subcore_barrier()

    pltpu.sync_copy(acc_sh.at[:, pl.ds(sub * NPW, NPW)], tmpv)

    @pl.loop(0, NPW // L)
    def _comb(j):
        s = tmpv[0, pl.ds(j * L, L)]
        for t in range(1, NS):
            s = s + tmpv[t, pl.ds(j * L, L)]
        resv[pl.ds(j * L, L)] = s

    pltpu.sync_copy(resv, out_hbm.at[core, pl.ds(sub * NPW, NPW)])


_hist = pl.kernel(
    _hist_body,
    out_type=jax.ShapeDtypeStruct((NC, NP), jnp.float32),
    mesh=MESH,
    compiler_params=pltpu.CompilerParams(needs_layout_passes=False),
    scratch_types=[
        pltpu.VMEM((NB_E, KB), jnp.int32),
        pltpu.VMEM((NP,), jnp.float32),
        pltpu.VMEM((NPW,), jnp.float32),
        pltpu.VMEM((NS, NPW), jnp.float32),
        pltpu.VMEM_SHARED((NS, NP), jnp.float32),
    ],
)


def _spmm_body(h_hbm, src_hbm, dst_hbm, out_hbm,
               srcv, dstv, gidx0, gidx1, rows0, rows1, zbuf, acc_sh,
               sem0, sem1):
    core = lax.axis_index("c")
    sub = lax.axis_index("s")
    gidx = [gidx0, gidx1]
    rows = [rows0, rows1]
    sems = [sem0, sem1]

    @pl.loop(0, L)
    def _z(i):
        for g in range(KB // L):
            zbuf[i, pl.ds(g * L, L)] = jnp.zeros((L,), jnp.float32)

    def _start_gather(b, i, off):
        for j in range(KB // L):
            gidx[b][pl.ds(j * L, L)] = srcv[i, pl.ds(j * L, L)] + off
        pltpu.async_copy(h_hbm.at[gidx[b]], rows[b], sems[b])

    for cc_local in range(2):
        cc = core * 2 + cc_local

        @pl.loop(0, NPW // L)
        def _zacc(j):
            pltpu.sync_copy(zbuf, acc_sh.at[pl.ds(sub * NPW + j * L, L)])

        plsc.subcore_barrier()

        for sl in range(2):
            pltpu.sync_copy(src_hbm.at[sub * 2 + sl], srcv)
            pltpu.sync_copy(dst_hbm.at[sub * 2 + sl], dstv)
            off = cc * NP
            _start_gather(0, 0, off)
            _start_gather(1, 1, off)

            @pl.loop(0, NB_E, step=2)
            def _batch(i):
                for b in range(2):
                    pltpu.make_async_copy(h_hbm.at[gidx[b]], rows[b],
                                          sems[b]).wait()
                    pltpu.sync_copy(rows[b], acc_sh.at[dstv.at[i + b]],
                                    add=True)

                    @pl.when(i + b + 2 < NB_E)
                    def _():
                        _start_gather(b, i + b + 2, off)

        plsc.subcore_barrier()
        pltpu.sync_copy(acc_sh.at[pl.ds(sub * NPW, NPW)],
                        out_hbm.at[cc, pl.ds(sub * NPW, NPW)])


_spmm = pl.kernel(
    _spmm_body,
    out_type=jax.ShapeDtypeStruct((CHUNKS, NP, 128), jnp.float32),
    mesh=MESH,
    compiler_params=pltpu.CompilerParams(needs_layout_passes=False),
    scratch_types=[
        pltpu.VMEM((NB_E, KB), jnp.int32),
        pltpu.VMEM((NB_E, KB), jnp.int32),
        pltpu.VMEM((KB,), jnp.int32),
        pltpu.VMEM((KB,), jnp.int32),
        pltpu.VMEM((KB, 128), jnp.float32),
        pltpu.VMEM((KB, 128), jnp.float32),
        pltpu.VMEM((L, 128), jnp.float32),
        pltpu.VMEM_SHARED((NP, 128), jnp.float32),
        pltpu.SemaphoreType.DMA,
        pltpu.SemaphoreType.DMA,
    ],
)


BN = 512


def _gemm1_body(x_ref, w_ref, dv_ref, o_ref):
    o_ref[0] = jnp.dot(x_ref[...], w_ref[...],
                       preferred_element_type=jnp.float32) * dv_ref[...]


def _gemm1(x, w, dinv_r):
    return pl.pallas_call(
        _gemm1_body,
        grid=(CHUNKS, NP // BN),
        in_specs=[
            pl.BlockSpec((BN, D_IN), lambda i, j: (j, 0)),
            pl.BlockSpec((D_IN, 128), lambda i, j: (0, i)),
            pl.BlockSpec((BN, 1), lambda i, j: (j, 0)),
        ],
        out_specs=pl.BlockSpec((1, BN, 128), lambda i, j: (i, j, 0)),
        out_shape=jax.ShapeDtypeStruct((CHUNKS, NP, 128), jnp.float32),
    )(x, w, dinv_r)


def _gemm2_body(a_ref, h_ref, b_ref, w_ref, dv_ref, o_ref):
    @pl.when(pl.program_id(2) == 0)
    def _():
        o_ref[0] = jnp.zeros_like(o_ref[0])

    act = jnp.maximum((a_ref[0] + h_ref[0]) * dv_ref[...] + b_ref[0], 0.0)
    o_ref[0] += jnp.dot(act, w_ref[...], preferred_element_type=jnp.float32)

    @pl.when(pl.program_id(2) == CHUNKS - 1)
    def _():
        o_ref[0] = o_ref[0] * dv_ref[...]


def _gemm2(agg, h, brow, w, dinv_r):
    return pl.pallas_call(
        _gemm2_body,
        grid=(CHUNKS, NP // BN, CHUNKS),
        in_specs=[
            pl.BlockSpec((1, BN, 128), lambda i, j, k: (k, j, 0)),
            pl.BlockSpec((1, BN, 128), lambda i, j, k: (k, j, 0)),
            pl.BlockSpec((1, 1, 128), lambda i, j, k: (k, 0, 0)),
            pl.BlockSpec((128, 128), lambda i, j, k: (k, i)),
            pl.BlockSpec((BN, 1), lambda i, j, k: (j, 0)),
        ],
        out_specs=pl.BlockSpec((1, BN, 128), lambda i, j, k: (i, j, 0)),
        out_shape=jax.ShapeDtypeStruct((CHUNKS, NP, 128), jnp.float32),
    )(agg, h, brow, w, dinv_r)


def _head_body(agg_ref, h_ref, dv_ref, b2_ref, wp1_ref, bp1_ref, wp2_ref,
               bp2_ref, o_ref):
    a = jnp.concatenate([agg_ref[c] + h_ref[c] for c in range(CHUNKS)],
                        axis=-1)
    a = jnp.maximum(a * dv_ref[...] + b2_ref[...], 0.0)
    p = jnp.maximum(jnp.dot(a, wp1_ref[...],
                            preferred_element_type=jnp.float32) + bp1_ref[...],
                    0.0)
    lg = jnp.dot(p, wp2_ref[...],
                 preferred_element_type=jnp.float32) + bp2_ref[...]
    m = jnp.max(lg, axis=-1, keepdims=True)
    ex = jnp.exp(lg - m)
    o_ref[...] = ex / jnp.sum(ex, axis=-1, keepdims=True)


def _head(agg, h, dinv_r, b2row, wp1, bp1row, wp2p, bp2p):
    return pl.pallas_call(
        _head_body,
        grid=(NP // BN,),
        in_specs=[
            pl.BlockSpec((CHUNKS, BN, 128), lambda j: (0, j, 0)),
            pl.BlockSpec((CHUNKS, BN, 128), lambda j: (0, j, 0)),
            pl.BlockSpec((BN, 1), lambda j: (j, 0)),
            pl.BlockSpec((1, D_H), lambda j: (0, 0)),
            pl.BlockSpec((D_H, D_H), lambda j: (0, 0)),
            pl.BlockSpec((1, D_H), lambda j: (0, 0)),
            pl.BlockSpec((D_H, 128), lambda j: (0, 0)),
            pl.BlockSpec((1, 128), lambda j: (0, 0)),
        ],
        out_specs=pl.BlockSpec((BN, 128), lambda j: (j, 0)),
        out_shape=jax.ShapeDtypeStruct((NP, 128), jnp.float32),
    )(agg, h, dinv_r, b2row, wp1, bp1row, wp2p, bp2p)


@jax.jit
def _run(x, edge_index, W1, b1, W2, b2, Wp1, bp1, Wp2, bp2):
    src = edge_index[0].astype(jnp.int32)
    dst = edge_index[1].astype(jnp.int32)
    padv = N + (jnp.arange(NW * (EW - ES), dtype=jnp.int32) % (NP - N))
    padv = padv.reshape(NW, EW - ES)
    src3 = jnp.concatenate([src.reshape(NW, ES), padv],
                           axis=1).reshape(NW, NB_E, KB)
    dst3 = jnp.concatenate([dst.reshape(NW, ES), padv],
                           axis=1).reshape(NW, NB_E, KB)

    degp = _hist(dst3)
    deg = degp[0] + degp[1] + 1.0
    dinv_r = lax.rsqrt(deg).reshape(NP, 1)

    x_pad = jnp.pad(x, ((0, NP - N), (0, 0)))
    h1 = _gemm1(x_pad, W1, dinv_r)
    agg1 = _spmm(h1.reshape(CHUNKS * NP, 128), src3, dst3)
    h2 = _gemm2(agg1, h1, b1.reshape(CHUNKS, 1, 128), W2, dinv_r)
    agg2 = _spmm(h2.reshape(CHUNKS * NP, 128), src3, dst3)

    wp2p = jnp.pad(Wp2, ((0, 0), (0, 128 - D_ACT)))
    bp2p = jnp.pad(bp2, (0, 128 - D_ACT), constant_values=-1e30)
    acts = _head(agg2, h2, dinv_r, b2.reshape(1, D_H), Wp1,
                 bp1.reshape(1, D_H),
                 wp2p, bp2p.reshape(1, 128))
    return acts[:N, :D_ACT]


def kernel(x, edge_index, W1, b1, W2, b2, Wp1, bp1, Wp2, bp2):
    return _run(x, edge_index, W1, b1, W2, b2, Wp1, bp1, Wp2, bp2)


# speedup vs baseline: 2.2248x; 1.1653x over previous
import jax
import jax.numpy as jnp
from jax import lax
from jax.experimental import pallas as pl
from jax.experimental.pallas import tpu as pltpu
from jax.experimental.pallas import tpu_sc as plsc

N = 10000
E = 160000
D_IN = 256
D_H = 512
D_ACT = 4

NP = 10240
NC, NS, L = 2, 16, 16
NW = NC * NS
KB = 128
NB_E = 40
EW = NB_E * KB
EP = EW * NW
ES = E // NW
NPW = NP // NS
CHUNKS = D_H // 128
MESH = plsc.VectorSubcoreMesh(core_axis_name="c", subcore_axis_name="s")


def _wid():
    return lax.axis_index("s") * NC + lax.axis_index("c")


def _hist_body(dst_hbm, out_hbm, dstv, histv, resv, tmpv, acc_sh):
    core = lax.axis_index("c")
    sub = lax.axis_index("s")
    wid = _wid()
    pltpu.sync_copy(dst_hbm.at[wid], dstv)

    @pl.loop(0, NP // L)
    def _zero(j):
        histv[pl.ds(j * L, L)] = jnp.zeros((L,), jnp.float32)

    ones = jnp.ones((L,), jnp.float32)
    lane = lax.iota(jnp.int32, L)

    @pl.loop(0, NB_E)
    def _batch(i):
        base = i * KB
        for j in range(KB // L):
            idx = dstv[i, pl.ds(j * L, L)]
            valid = (base + j * L + lane) < ES
            plsc.addupdate_scatter(histv, [idx], ones, mask=valid)

    pltpu.sync_copy(histv, acc_sh.at[sub])
    plsc.subcore_barrier()

    pltpu.sync_copy(acc_sh.at[:, pl.ds(sub * NPW, NPW)], tmpv)

    @pl.loop(0, NPW // L)
    def _comb(j):
        s = tmpv[0, pl.ds(j * L, L)]
        for t in range(1, NS):
            s = s + tmpv[t, pl.ds(j * L, L)]
        resv[pl.ds(j * L, L)] = s

    pltpu.sync_copy(resv, out_hbm.at[core, pl.ds(sub * NPW, NPW)])


_hist = pl.kernel(
    _hist_body,
    out_type=jax.ShapeDtypeStruct((NC, NP), jnp.float32),
    mesh=MESH,
    compiler_params=pltpu.CompilerParams(needs_layout_passes=False),
    scratch_types=[
        pltpu.VMEM((NB_E, KB), jnp.int32),
        pltpu.VMEM((NP,), jnp.float32),
        pltpu.VMEM((NPW,), jnp.float32),
        pltpu.VMEM((NS, NPW), jnp.float32),
        pltpu.VMEM_SHARED((NS, NP), jnp.float32),
    ],
)


def _spmm_body(h_hbm, src_hbm, dst_hbm, out_hbm,
               srcv, dstv, gidx0, gidx1, rows0, rows1, zbuf, acc_sh,
               sem0, sem1):
    core = lax.axis_index("c")
    sub = lax.axis_index("s")
    gidx = [gidx0, gidx1]
    rows = [rows0, rows1]
    sems = [sem0, sem1]

    @pl.loop(0, L)
    def _z(i):
        for g in range(KB // L):
            zbuf[i, pl.ds(g * L, L)] = jnp.zeros((L,), jnp.float32)

    def _start_gather(b, i, off):
        for j in range(KB // L):
            gidx[b][pl.ds(j * L, L)] = srcv[i, pl.ds(j * L, L)] + off
        pltpu.async_copy(h_hbm.at[gidx[b]], rows[b], sems[b])

    for cc_local in range(2):
        cc = core * 2 + cc_local

        @pl.loop(0, NPW // L)
        def _zacc(j):
            pltpu.sync_copy(zbuf, acc_sh.at[pl.ds(sub * NPW + j * L, L)])

        plsc.subcore_barrier()

        for sl in range(2):
            pltpu.sync_copy(src_hbm.at[sub * 2 + sl], srcv)
            pltpu.sync_copy(dst_hbm.at[sub * 2 + sl], dstv)
            off = cc * NP
            _start_gather(0, 0, off)
            _start_gather(1, 1, off)

            @pl.loop(0, NB_E, step=2)
            def _batch(i):
                for b in range(2):
                    pltpu.make_async_copy(h_hbm.at[gidx[b]], rows[b],
                                          sems[b]).wait()
                    pltpu.sync_copy(rows[b], acc_sh.at[dstv.at[i + b]],
                                    add=True)

                    @pl.when(i + b + 2 < NB_E)
                    def _():
                        _start_gather(b, i + b + 2, off)

        plsc.subcore_barrier()
        pltpu.sync_copy(acc_sh.at[pl.ds(sub * NPW, NPW)],
                        out_hbm.at[cc, pl.ds(sub * NPW, NPW)])


_spmm = pl.kernel(
    _spmm_body,
    out_type=jax.ShapeDtypeStruct((CHUNKS, NP, 128), jnp.float32),
    mesh=MESH,
    compiler_params=pltpu.CompilerParams(needs_layout_passes=False),
    scratch_types=[
        pltpu.VMEM((NB_E, KB), jnp.int32),
        pltpu.VMEM((NB_E, KB), jnp.int32),
        pltpu.VMEM((KB,), jnp.int32),
        pltpu.VMEM((KB,), jnp.int32),
        pltpu.VMEM((KB, 128), jnp.float32),
        pltpu.VMEM((KB, 128), jnp.float32),
        pltpu.VMEM((L, 128), jnp.float32),
        pltpu.VMEM_SHARED((NP, 128), jnp.float32),
        pltpu.SemaphoreType.DMA,
        pltpu.SemaphoreType.DMA,
    ],
)


BN = 1024


def _gemm1_body(x_ref, w_ref, dv_ref, o_ref):
    o_ref[0] = jnp.dot(x_ref[...], w_ref[...],
                       preferred_element_type=jnp.float32) * dv_ref[...]


def _gemm1(x, w, dinv_r):
    return pl.pallas_call(
        _gemm1_body,
        grid=(CHUNKS, NP // BN),
        in_specs=[
            pl.BlockSpec((BN, D_IN), lambda i, j: (j, 0)),
            pl.BlockSpec((D_IN, 128), lambda i, j: (0, i)),
            pl.BlockSpec((BN, 1), lambda i, j: (j, 0)),
        ],
        out_specs=pl.BlockSpec((1, BN, 128), lambda i, j: (i, j, 0)),
        out_shape=jax.ShapeDtypeStruct((CHUNKS, NP, 128), jnp.float32),
    )(x, w, dinv_r)


def _gemm2_body(a_ref, h_ref, b_ref, w_ref, dv_ref, o_ref):
    @pl.when(pl.program_id(2) == 0)
    def _():
        o_ref[0] = jnp.zeros_like(o_ref[0])

    act = jnp.maximum((a_ref[0] + h_ref[0]) * dv_ref[...] + b_ref[0], 0.0)
    o_ref[0] += jnp.dot(act, w_ref[...], preferred_element_type=jnp.float32)

    @pl.when(pl.program_id(2) == CHUNKS - 1)
    def _():
        o_ref[0] = o_ref[0] * dv_ref[...]


def _gemm2(agg, h, brow, w, dinv_r):
    return pl.pallas_call(
        _gemm2_body,
        grid=(CHUNKS, NP // BN, CHUNKS),
        in_specs=[
            pl.BlockSpec((1, BN, 128), lambda i, j, k: (k, j, 0)),
            pl.BlockSpec((1, BN, 128), lambda i, j, k: (k, j, 0)),
            pl.BlockSpec((1, 1, 128), lambda i, j, k: (k, 0, 0)),
            pl.BlockSpec((128, 128), lambda i, j, k: (k, i)),
            pl.BlockSpec((BN, 1), lambda i, j, k: (j, 0)),
        ],
        out_specs=pl.BlockSpec((1, BN, 128), lambda i, j, k: (i, j, 0)),
        out_shape=jax.ShapeDtypeStruct((CHUNKS, NP, 128), jnp.float32),
    )(agg, h, brow, w, dinv_r)


def _head_body(agg_ref, h_ref, dv_ref, b2_ref, wp1_ref, bp1_ref, wp2_ref,
               bp2_ref, o_ref):
    a = jnp.concatenate([agg_ref[c] + h_ref[c] for c in range(CHUNKS)],
                        axis=-1)
    a = jnp.maximum(a * dv_ref[...] + b2_ref[...], 0.0)
    p = jnp.maximum(jnp.dot(a, wp1_ref[...],
                            preferred_element_type=jnp.float32) + bp1_ref[...],
                    0.0)
    lg = jnp.dot(p, wp2_ref[...],
                 preferred_element_type=jnp.float32) + bp2_ref[...]
    m = jnp.max(lg, axis=-1, keepdims=True)
    ex = jnp.exp(lg - m)
    o_ref[...] = ex / jnp.sum(ex, axis=-1, keepdims=True)


def _head(agg, h, dinv_r, b2row, wp1, bp1row, wp2p, bp2p):
    return pl.pallas_call(
        _head_body,
        grid=(NP // BN,),
        in_specs=[
            pl.BlockSpec((CHUNKS, BN, 128), lambda j: (0, j, 0)),
            pl.BlockSpec((CHUNKS, BN, 128), lambda j: (0, j, 0)),
            pl.BlockSpec((BN, 1), lambda j: (j, 0)),
            pl.BlockSpec((1, D_H), lambda j: (0, 0)),
            pl.BlockSpec((D_H, D_H), lambda j: (0, 0)),
            pl.BlockSpec((1, D_H), lambda j: (0, 0)),
            pl.BlockSpec((D_H, 128), lambda j: (0, 0)),
            pl.BlockSpec((1, 128), lambda j: (0, 0)),
        ],
        out_specs=pl.BlockSpec((BN, 128), lambda j: (j, 0)),
        out_shape=jax.ShapeDtypeStruct((NP, 128), jnp.float32),
    )(agg, h, dinv_r, b2row, wp1, bp1row, wp2p, bp2p)


@jax.jit
def _run(x, edge_index, W1, b1, W2, b2, Wp1, bp1, Wp2, bp2):
    src = edge_index[0].astype(jnp.int32)
    dst = edge_index[1].astype(jnp.int32)
    padv = N + (jnp.arange(NW * (EW - ES), dtype=jnp.int32) % (NP - N))
    padv = padv.reshape(NW, EW - ES)
    src3 = jnp.concatenate([src.reshape(NW, ES), padv],
                           axis=1).reshape(NW, NB_E, KB)
    dst3 = jnp.concatenate([dst.reshape(NW, ES), padv],
                           axis=1).reshape(NW, NB_E, KB)

    degp = _hist(dst3)
    deg = degp[0] + degp[1] + 1.0
    dinv_r = lax.rsqrt(deg).reshape(NP, 1)

    x_pad = jnp.pad(x, ((0, NP - N), (0, 0)))
    h1 = _gemm1(x_pad, W1, dinv_r)
    agg1 = _spmm(h1.reshape(CHUNKS * NP, 128), src3, dst3)
    h2 = _gemm2(agg1, h1, b1.reshape(CHUNKS, 1, 128), W2, dinv_r)
    agg2 = _spmm(h2.reshape(CHUNKS * NP, 128), src3, dst3)

    wp2p = jnp.pad(Wp2, ((0, 0), (0, 128 - D_ACT)))
    bp2p = jnp.pad(bp2, (0, 128 - D_ACT), constant_values=-1e30)
    acts = _head(agg2, h2, dinv_r, b2.reshape(1, D_H), Wp1,
                 bp1.reshape(1, D_H),
                 wp2p, bp2p.reshape(1, 128))
    return acts[:N, :D_ACT]


def kernel(x, edge_index, W1, b1, W2, b2, Wp1, bp1, Wp2, bp2):
    return _run(x, edge_index, W1, b1, W2, b2, Wp1, bp1, Wp2, bp2)
